# Initial kernel scaffold; baseline (speedup 1.0000x reference)
import jax
import jax.numpy as jnp
from jax.experimental import pallas as pl


def kernel(x, edge_index, W_time, W_space):
    raise NotImplementedError("write your pallas kernel here")


# speedup vs baseline: 22.1328x; 22.1328x over previous
import functools

import jax
import jax.numpy as jnp
from jax import lax
from jax.experimental import pallas as pl
from jax.experimental.pallas import tpu as pltpu
from jax.experimental.pallas import tpu_sc as plsc

N = 10000
E = 320000
T = 7
S = 121
EPS = 1e-5
MAXN = 1e6
D = 130
DP = 144
NP = 10240
NC = 2
NSC = 16
NW = NC * NSC
EPW = E // NW
CH = 80
NCHUNK = EPW // CH
RPT = NP // NSC
ZB = RPT // CH


def _deg_body(rows_hbm, zo_hbm, out_hbm, idx2d, zeros_v, ones_v, acc):
    cid = lax.axis_index("c")
    sid = lax.axis_index("s")
    wid = cid * NSC + sid
    pltpu.sync_copy(zo_hbm.at[0], zeros_v)
    pltpu.sync_copy(zo_hbm.at[1], ones_v)
    pltpu.sync_copy(rows_hbm.at[wid], idx2d)

    def zbody(t, carry):
        pltpu.sync_copy(zeros_v, acc.at[pl.ds(sid * RPT + t * CH, CH)])
        return carry

    lax.fori_loop(0, ZB, zbody, 0)
    plsc.subcore_barrier()

    def body(j, carry):
        pltpu.sync_copy(ones_v, acc.at[idx2d.at[j]], add=True)
        return carry

    lax.fori_loop(0, NCHUNK, body, 0)
    plsc.subcore_barrier()
    pltpu.sync_copy(acc.at[pl.ds(sid * RPT, RPT)],
                    out_hbm.at[cid, pl.ds(sid * RPT, RPT)])


def _deg_kernel(rows3d, zo):
    mesh = plsc.VectorSubcoreMesh(core_axis_name="c", subcore_axis_name="s")
    return pl.kernel(
        _deg_body,
        mesh=mesh,
        out_type=jax.ShapeDtypeStruct((NC, NP), jnp.float32),
        scratch_types=[
            pltpu.VMEM((NCHUNK, CH), jnp.int32),
            pltpu.VMEM((CH,), jnp.float32),
            pltpu.VMEM((CH,), jnp.float32),
            pltpu.VMEM_SHARED((NP,), jnp.float32),
        ],
    )(rows3d, zo)


def _agg_body(sh_hbm, cols_hbm, rows_hbm, zer_hbm, out_hbm,
              col2d, row2d, rbuf, acc, sem):
    cid = lax.axis_index("c")
    sid = lax.axis_index("s")
    wid = cid * NSC + sid
    pltpu.sync_copy(cols_hbm.at[wid], col2d)
    pltpu.sync_copy(rows_hbm.at[wid], row2d)
    pltpu.sync_copy(zer_hbm, rbuf)

    def zbody(t, carry):
        pltpu.sync_copy(rbuf, acc.at[pl.ds(sid * RPT + t * CH, CH)])
        return carry

    lax.fori_loop(0, ZB, zbody, 0)
    plsc.subcore_barrier()

    def body(j, carry):
        pltpu.async_copy(sh_hbm.at[col2d.at[j]], rbuf, sem).wait()
        pltpu.sync_copy(rbuf, acc.at[row2d.at[j]], add=True)
        return carry

    lax.fori_loop(0, NCHUNK, body, 0)
    plsc.subcore_barrier()
    pltpu.sync_copy(acc.at[pl.ds(sid * RPT, RPT)],
                    out_hbm.at[cid, pl.ds(sid * RPT, RPT)])


def _agg_kernel(sh2, cols3d, rows3d, zer):
    mesh = plsc.VectorSubcoreMesh(core_axis_name="c", subcore_axis_name="s")
    return pl.kernel(
        _agg_body,
        mesh=mesh,
        out_type=jax.ShapeDtypeStruct((NC, NP, DP), jnp.float32),
        scratch_types=[
            pltpu.VMEM((NCHUNK, CH), jnp.int32),
            pltpu.VMEM((NCHUNK, CH), jnp.int32),
            pltpu.VMEM((CH, DP), jnp.float32),
            pltpu.VMEM_SHARED((NP, DP), jnp.float32),
            pltpu.SemaphoreType.DMA,
        ],
        compiler_params=pltpu.CompilerParams(use_tc_tiling_on_sc=False),
    )(sh2, cols3d, rows3d, zer)


def _sphere_fix(v):
    n = jnp.sqrt(jnp.sum(v * v, axis=1, keepdims=True)) + EPS
    mask = (n > MAXN).astype(v.dtype)
    nc = jnp.minimum(n, MAXN)
    v_ = v / nc
    v_ = v_ * mask + v * (1 - mask)
    vn = jnp.sqrt(jnp.sum(v_ * v_, axis=1, keepdims=True))
    return v_ / jnp.clip(vn, 1e-12, None)


BN = 1000


def _stage1_body(x_ref, wb_ref, dg_ref, o_ref):
    xb = x_ref[...]
    wb = wb_ref[...]
    dg = dg_ref[...]
    deg = dg[:, 0:1] + dg[:, 1:2] + 1.0
    dis = lax.rsqrt(deg)

    u_t = xb[:, :T]
    u_sp = xb[:, T:]
    st2 = jnp.sum(u_t * u_t, axis=1, keepdims=True)
    ssp2 = jnp.sum(u_sp * u_sp, axis=1, keepdims=True)
    x0 = jnp.sqrt(jnp.clip(1.0 + ssp2 - st2, EPS, None))
    nt = jnp.sqrt(x0 * x0 + st2) + EPS
    f = jnp.concatenate(
        [x0 / nt, u_t / nt, u_sp,
         jnp.zeros((xb.shape[0], DP - 129), jnp.float32)], axis=1)
    fm = jnp.dot(f, wb, preferred_element_type=jnp.float32)
    s = fm[:, :T + 1]
    h = fm[:, T + 1:129]

    s_ = _sphere_fix(s)
    h2 = jnp.sum(h * h, axis=1, keepdims=True)
    h0 = jnp.sqrt(h2 + 1.0)

    gt = s_[:, 1:] * h0
    gt2 = jnp.sum(gt * gt, axis=1, keepdims=True)
    y0 = jnp.sqrt(jnp.clip(1.0 + h2 - gt2, EPS, None))
    nt2 = jnp.sqrt(y0 * y0 + gt2) + EPS
    out = jnp.concatenate(
        [y0 / nt2, gt / nt2, nt2, h,
         jnp.zeros((xb.shape[0], DP - D), jnp.float32)], axis=1)
    o_ref[...] = dis * out


def _stage1(x, wb, degp):
    grid = (N // BN,)
    return pl.pallas_call(
        _stage1_body,
        grid=grid,
        in_specs=[
            pl.BlockSpec((BN, 128), lambda i: (i, 0)),
            pl.BlockSpec((DP, DP), lambda i: (0, 0)),
            pl.BlockSpec((BN, NC), lambda i: (i, 0)),
        ],
        out_specs=pl.BlockSpec((BN, DP), lambda i: (i, 0)),
        out_shape=jax.ShapeDtypeStruct((N, DP), jnp.float32),
    )(x, wb, degp)


def _stage2_body(pp_ref, sh_ref, dg_ref, o_ref):
    pp = pp_ref[...]
    shr = sh_ref[...]
    dg = dg_ref[...]
    deg = dg[:, 0:1] + dg[:, 1:2] + 1.0
    dis = lax.rsqrt(deg)

    u = dis * (pp[0] + pp[1] + shr)
    s = u[:, :T + 1]
    s_ = _sphere_fix(s)

    ah0 = u[:, T + 1:T + 2]
    ahs = u[:, T + 2:D]
    mink = jnp.sum(ahs * ahs, axis=1, keepdims=True) - ah0 * ah0
    n = jnp.sqrt(jnp.abs(mink) + EPS) + EPS
    nc = jnp.minimum(n, MAXN)
    b0 = ah0 / nc
    bs = ahs / nc
    mink2 = jnp.sum(bs * bs, axis=1, keepdims=True) - b0 * b0
    n2 = jnp.sqrt(jnp.abs(mink2) + EPS) + EPS
    b0 = b0 / n2
    bs = bs / n2

    wt = s_[:, 1:] * b0
    wsp = bs

    wt2 = jnp.sum(wt * wt, axis=1, keepdims=True)
    wsp2 = jnp.sum(wsp * wsp, axis=1, keepdims=True)
    z0 = jnp.sqrt(jnp.clip(1.0 + wsp2 - wt2, EPS, None))
    ntz = jnp.sqrt(z0 * z0 + wt2) + EPS
    sz = jnp.concatenate([z0 / ntz, wt / ntz], axis=1)
    s2 = jnp.maximum(sz, 0.0)
    hsp = jnp.maximum(wsp, 0.0)

    s2_ = _sphere_fix(s2)
    hn = jnp.sqrt(jnp.sum(hsp * hsp, axis=1, keepdims=True)) + EPS
    maskh = (hn > MAXN).astype(jnp.float32)
    hnc = jnp.minimum(hn, MAXN)
    t = hsp / hnc
    tn = jnp.sqrt(jnp.sum(t * t, axis=1, keepdims=True))
    h_ = t / jnp.clip(tn, 1e-12, None) * MAXN
    hspf = h_ * maskh + hsp * (1 - maskh)
    h0z = jnp.sqrt(hnc * hnc + 1.0)

    o_ref[...] = jnp.concatenate([s2_[:, 1:] * h0z, hspf], axis=1)


def _stage2(parts, sh2, degp):
    grid = (N // BN,)
    return pl.pallas_call(
        _stage2_body,
        grid=grid,
        in_specs=[
            pl.BlockSpec((NC, BN, DP), lambda i: (0, i, 0)),
            pl.BlockSpec((BN, DP), lambda i: (i, 0)),
            pl.BlockSpec((BN, NC), lambda i: (i, 0)),
        ],
        out_specs=pl.BlockSpec((BN, 128), lambda i: (i, 0)),
        out_shape=jax.ShapeDtypeStruct((N, 128), jnp.float32),
    )(parts, sh2, degp)


def kernel(x, edge_index, W_time, W_space):
    rows3d = edge_index[0].reshape(NW, NCHUNK, CH)
    cols3d = edge_index[1].reshape(NW, NCHUNK, CH)
    zo = jnp.stack([jnp.zeros((CH,), jnp.float32),
                    jnp.ones((CH,), jnp.float32)])
    zer = jnp.zeros((CH, DP), jnp.float32)
    wb = jnp.zeros((DP, DP), jnp.float32)
    wb = wb.at[:T + 1, :T + 1].set(W_time)
    wb = wb.at[T + 1:129, T + 1:129].set(W_space)

    degp = _deg_kernel(rows3d, zo)
    dgn = degp[:, :N].T
    sh2 = _stage1(x, wb, dgn)
    parts = _agg_kernel(sh2, cols3d, rows3d, zer)
    return _stage2(parts[:, :N], sh2, dgn)


# speedup vs baseline: 27.4756x; 1.2414x over previous
import functools

import jax
import jax.numpy as jnp
from jax import lax
from jax.experimental import pallas as pl
from jax.experimental.pallas import tpu as pltpu
from jax.experimental.pallas import tpu_sc as plsc

N = 10000
E = 320000
T = 7
S = 121
EPS = 1e-5
MAXN = 1e6
D = 130
DP = 144
NP = 10240
NC = 2
NSC = 16
NW = NC * NSC
EPW = E // NW
CH = 40
NCHUNK = EPW // CH
RPT = NP // NSC
ZB = RPT // CH


def _deg_body(eidx_hbm, zo_hbm, out_hbm, idx3, zeros_v, ones_v, acc):
    cid = lax.axis_index("c")
    sid = lax.axis_index("s")
    wid = cid * NSC + sid
    pltpu.sync_copy(zo_hbm.at[0], zeros_v)
    pltpu.sync_copy(zo_hbm.at[1], ones_v)
    pltpu.sync_copy(eidx_hbm.at[wid], idx3)

    def zbody(t, carry):
        pltpu.sync_copy(zeros_v, acc.at[pl.ds(sid * RPT + t * CH, CH)])
        return carry

    lax.fori_loop(0, ZB, zbody, 0)
    plsc.subcore_barrier()

    def body(j, carry):
        pltpu.sync_copy(ones_v, acc.at[idx3.at[j, 0]], add=True)
        return carry

    lax.fori_loop(0, NCHUNK, body, 0)
    plsc.subcore_barrier()
    pltpu.sync_copy(acc.at[pl.ds(sid * RPT, RPT)],
                    out_hbm.at[cid, pl.ds(sid * RPT, RPT)])


def _deg_kernel(eidx, zo):
    mesh = plsc.VectorSubcoreMesh(core_axis_name="c", subcore_axis_name="s")
    return pl.kernel(
        _deg_body,
        mesh=mesh,
        out_type=jax.ShapeDtypeStruct((NC, NP), jnp.float32),
        scratch_types=[
            pltpu.VMEM((NCHUNK, 2, CH), jnp.int32),
            pltpu.VMEM((CH,), jnp.float32),
            pltpu.VMEM((CH,), jnp.float32),
            pltpu.VMEM_SHARED((NP,), jnp.float32),
        ],
        compiler_params=pltpu.CompilerParams(use_tc_tiling_on_sc=False),
    )(eidx, zo)


NB = 5
NI = 10


def _agg_body(sh_hbm, eidx_hbm, zer_hbm, out_hbm,
              rb0, rb1, rb2, rb3, rb4,
              ib0, ib1, ib2, ib3, ib4, ib5, ib6, ib7, ib8, ib9,
              acc, gsem, ssem, isem):
    rb = (rb0, rb1, rb2, rb3, rb4)
    ib = (ib0, ib1, ib2, ib3, ib4, ib5, ib6, ib7, ib8, ib9)
    cid = lax.axis_index("c")
    sid = lax.axis_index("s")
    wid = cid * NSC + sid

    pltpu.sync_copy(zer_hbm, rb0)
    for t in range(ZB):
        pltpu.sync_copy(rb0, acc.at[pl.ds(sid * RPT + t * CH, CH)])
    plsc.subcore_barrier()

    def istart(m, s):
        pltpu.async_copy(eidx_hbm.at[wid, m], ib[s], isem.at[s])

    def iwait(m, s):
        pltpu.make_async_copy(eidx_hbm.at[wid, m], ib[s], isem.at[s]).wait()

    def gstart(s, si):
        pltpu.async_copy(sh_hbm.at[ib[si].at[1]], rb[s], gsem.at[s])

    def gwait(s, si):
        pltpu.make_async_copy(sh_hbm.at[ib[si].at[1]], rb[s], gsem.at[s]).wait()

    def sstart(s, si):
        pltpu.async_copy(rb[s], acc.at[ib[si].at[0]], ssem.at[s], add=True)

    def swait(s, si):
        pltpu.make_async_copy(rb[s], acc.at[ib[si].at[0]], ssem.at[s]).wait()

    def step(m, b, has_istart, has_g, has_swait):
        s, si = b % NB, b % NI
        if has_istart:
            istart(m + 4, (b + 4) % NI)
        if has_g:
            iwait(m + 2, (b + 2) % NI)
            if has_swait:
                swait((b + 2) % NB, (b - 3) % NI)
            gstart((b + 2) % NB, (b + 2) % NI)
        gwait(s, si)
        sstart(s, si)

    for m in range(4):
        istart(m, m)
    iwait(0, 0)
    gstart(0, 0)
    iwait(1, 1)
    gstart(1, 1)
    for m in range(10):
        step(m, m, has_istart=(m + 4 <= NCHUNK - 1), has_g=True,
             has_swait=(m >= 3))

    def kbody(k, carry):
        for b in range(10):
            step(10 * k + b, b, True, True, True)
        return carry

    lax.fori_loop(1, NCHUNK // 10 - 1, kbody, 0)

    base = NCHUNK - 10
    for b in range(10):
        m = base + b
        step(m, b, has_istart=(m + 4 <= NCHUNK - 1),
             has_g=(m + 2 <= NCHUNK - 1), has_swait=True)
    for m in range(NCHUNK - NB, NCHUNK):
        swait(m % NB, m % NI)

    plsc.subcore_barrier()
    pltpu.sync_copy(acc.at[pl.ds(sid * RPT, RPT)],
                    out_hbm.at[cid, pl.ds(sid * RPT, RPT)])


def _agg_kernel(sh2, eidx, zer):
    mesh = plsc.VectorSubcoreMesh(core_axis_name="c", subcore_axis_name="s")
    return pl.kernel(
        _agg_body,
        mesh=mesh,
        out_type=jax.ShapeDtypeStruct((NC, NP, DP), jnp.float32),
        scratch_types=(
            [pltpu.VMEM((CH, DP), jnp.float32)] * NB
            + [pltpu.VMEM((2, CH), jnp.int32)] * NI
            + [
                pltpu.VMEM_SHARED((NP, DP), jnp.float32),
                pltpu.SemaphoreType.DMA((NB,)),
                pltpu.SemaphoreType.DMA((NB,)),
                pltpu.SemaphoreType.DMA((NI,)),
            ]
        ),
        compiler_params=pltpu.CompilerParams(use_tc_tiling_on_sc=False),
    )(sh2, eidx, zer)


def _sphere_fix(v):
    n = jnp.sqrt(jnp.sum(v * v, axis=1, keepdims=True)) + EPS
    mask = (n > MAXN).astype(v.dtype)
    nc = jnp.minimum(n, MAXN)
    v_ = v / nc
    v_ = v_ * mask + v * (1 - mask)
    vn = jnp.sqrt(jnp.sum(v_ * v_, axis=1, keepdims=True))
    return v_ / jnp.clip(vn, 1e-12, None)


BN = 1000


def _stage1_body(x_ref, wb_ref, dg_ref, o_ref):
    xb = x_ref[...]
    wb = wb_ref[...]
    dg = dg_ref[...]
    deg = dg[:, 0:1] + dg[:, 1:2] + 1.0
    dis = lax.rsqrt(deg)

    u_t = xb[:, :T]
    u_sp = xb[:, T:]
    st2 = jnp.sum(u_t * u_t, axis=1, keepdims=True)
    ssp2 = jnp.sum(u_sp * u_sp, axis=1, keepdims=True)
    x0 = jnp.sqrt(jnp.clip(1.0 + ssp2 - st2, EPS, None))
    nt = jnp.sqrt(x0 * x0 + st2) + EPS
    f = jnp.concatenate(
        [x0 / nt, u_t / nt, u_sp,
         jnp.zeros((xb.shape[0], DP - 129), jnp.float32)], axis=1)
    fm = jnp.dot(f, wb, preferred_element_type=jnp.float32)
    s = fm[:, :T + 1]
    h = fm[:, T + 1:129]

    s_ = _sphere_fix(s)
    h2 = jnp.sum(h * h, axis=1, keepdims=True)
    h0 = jnp.sqrt(h2 + 1.0)

    gt = s_[:, 1:] * h0
    gt2 = jnp.sum(gt * gt, axis=1, keepdims=True)
    y0 = jnp.sqrt(jnp.clip(1.0 + h2 - gt2, EPS, None))
    nt2 = jnp.sqrt(y0 * y0 + gt2) + EPS
    out = jnp.concatenate(
        [y0 / nt2, gt / nt2, nt2, h,
         jnp.zeros((xb.shape[0], DP - D), jnp.float32)], axis=1)
    o_ref[...] = dis * out


def _stage1(x, wb, degp):
    grid = (N // BN,)
    return pl.pallas_call(
        _stage1_body,
        grid=grid,
        in_specs=[
            pl.BlockSpec((BN, 128), lambda i: (i, 0)),
            pl.BlockSpec((DP, DP), lambda i: (0, 0)),
            pl.BlockSpec((BN, NC), lambda i: (i, 0)),
        ],
        out_specs=pl.BlockSpec((BN, DP), lambda i: (i, 0)),
        out_shape=jax.ShapeDtypeStruct((N, DP), jnp.float32),
    )(x, wb, degp)


def _stage2_body(pp_ref, sh_ref, dg_ref, o_ref):
    pp = pp_ref[...]
    shr = sh_ref[...]
    dg = dg_ref[...]
    deg = dg[:, 0:1] + dg[:, 1:2] + 1.0
    dis = lax.rsqrt(deg)

    u = dis * (pp[0] + pp[1] + shr)
    s = u[:, :T + 1]
    s_ = _sphere_fix(s)

    ah0 = u[:, T + 1:T + 2]
    ahs = u[:, T + 2:D]
    mink = jnp.sum(ahs * ahs, axis=1, keepdims=True) - ah0 * ah0
    n = jnp.sqrt(jnp.abs(mink) + EPS) + EPS
    nc = jnp.minimum(n, MAXN)
    b0 = ah0 / nc
    bs = ahs / nc
    mink2 = jnp.sum(bs * bs, axis=1, keepdims=True) - b0 * b0
    n2 = jnp.sqrt(jnp.abs(mink2) + EPS) + EPS
    b0 = b0 / n2
    bs = bs / n2

    wt = s_[:, 1:] * b0
    wsp = bs

    wt2 = jnp.sum(wt * wt, axis=1, keepdims=True)
    wsp2 = jnp.sum(wsp * wsp, axis=1, keepdims=True)
    z0 = jnp.sqrt(jnp.clip(1.0 + wsp2 - wt2, EPS, None))
    ntz = jnp.sqrt(z0 * z0 + wt2) + EPS
    sz = jnp.concatenate([z0 / ntz, wt / ntz], axis=1)
    s2 = jnp.maximum(sz, 0.0)
    hsp = jnp.maximum(wsp, 0.0)

    s2_ = _sphere_fix(s2)
    hn = jnp.sqrt(jnp.sum(hsp * hsp, axis=1, keepdims=True)) + EPS
    maskh = (hn > MAXN).astype(jnp.float32)
    hnc = jnp.minimum(hn, MAXN)
    t = hsp / hnc
    tn = jnp.sqrt(jnp.sum(t * t, axis=1, keepdims=True))
    h_ = t / jnp.clip(tn, 1e-12, None) * MAXN
    hspf = h_ * maskh + hsp * (1 - maskh)
    h0z = jnp.sqrt(hnc * hnc + 1.0)

    o_ref[...] = jnp.concatenate([s2_[:, 1:] * h0z, hspf], axis=1)


def _stage2(parts, sh2, degp):
    grid = (N // BN,)
    return pl.pallas_call(
        _stage2_body,
        grid=grid,
        in_specs=[
            pl.BlockSpec((NC, BN, DP), lambda i: (0, i, 0)),
            pl.BlockSpec((BN, DP), lambda i: (i, 0)),
            pl.BlockSpec((BN, NC), lambda i: (i, 0)),
        ],
        out_specs=pl.BlockSpec((BN, 128), lambda i: (i, 0)),
        out_shape=jax.ShapeDtypeStruct((N, 128), jnp.float32),
    )(parts, sh2, degp)


def kernel(x, edge_index, W_time, W_space):
    eidx = edge_index.reshape(2, NW, NCHUNK, CH).transpose(1, 2, 0, 3)
    zo = jnp.stack([jnp.zeros((CH,), jnp.float32),
                    jnp.ones((CH,), jnp.float32)])
    zer = jnp.zeros((CH, DP), jnp.float32)
    wb = jnp.zeros((DP, DP), jnp.float32)
    wb = wb.at[:T + 1, :T + 1].set(W_time)
    wb = wb.at[T + 1:129, T + 1:129].set(W_space)

    degp = _deg_kernel(eidx, zo)
    dgn = degp[:, :N].T
    sh2 = _stage1(x, wb, dgn)
    parts = _agg_kernel(sh2, eidx, zer)
    return _stage2(parts, sh2, dgn)


# speedup vs baseline: 32.4150x; 1.1798x over previous
import functools

import jax
import jax.numpy as jnp
from jax import lax
from jax.experimental import pallas as pl
from jax.experimental.pallas import tpu as pltpu
from jax.experimental.pallas import tpu_sc as plsc

N = 10000
E = 320000
T = 7
S = 121
EPS = 1e-5
MAXN = 1e6
D = 130
DP = 144
NP = 10240
NC = 2
NSC = 16
NW = NC * NSC
EPW = E // NW
CH = 40
NCHUNK = EPW // CH
RPT = NP // NSC
ZB = RPT // CH


def _deg_body(rows_hbm, zo_hbm, out_hbm, idx2, zeros_v, ones_v, acc, ssem):
    cid = lax.axis_index("c")
    sid = lax.axis_index("s")
    wid = cid * NSC + sid
    pltpu.sync_copy(zo_hbm.at[0], zeros_v)
    pltpu.sync_copy(zo_hbm.at[1], ones_v)
    pltpu.sync_copy(rows_hbm.at[wid], idx2)

    def zbody(t, carry):
        pltpu.sync_copy(zeros_v, acc.at[pl.ds(sid * RPT + t * CH, CH)])
        return carry

    lax.fori_loop(0, ZB, zbody, 0)
    plsc.subcore_barrier()

    def sstart(j, s):
        pltpu.async_copy(ones_v, acc.at[idx2.at[j]], ssem.at[s], add=True)

    def swait(j, s):
        pltpu.make_async_copy(ones_v, acc.at[idx2.at[j]], ssem.at[s]).wait()

    for b in range(10):
        sstart(b, b)

    def body(k, carry):
        for b in range(10):
            j = 10 * k + b
            swait(j, b)
            sstart(j + 10, b)
        return carry

    lax.fori_loop(0, NCHUNK // 10 - 1, body, 0)
    for b in range(10):
        swait(NCHUNK - 10 + b, b)

    plsc.subcore_barrier()
    pltpu.sync_copy(acc.at[pl.ds(sid * RPT, RPT)],
                    out_hbm.at[cid, pl.ds(sid * RPT, RPT)])


def _deg_kernel(rows3d, zo):
    mesh = plsc.VectorSubcoreMesh(core_axis_name="c", subcore_axis_name="s")
    return pl.kernel(
        _deg_body,
        mesh=mesh,
        out_type=jax.ShapeDtypeStruct((NC, NP), jnp.float32),
        scratch_types=[
            pltpu.VMEM((NCHUNK, CH), jnp.int32),
            pltpu.VMEM((CH,), jnp.float32),
            pltpu.VMEM((CH,), jnp.float32),
            pltpu.VMEM_SHARED((NP,), jnp.float32),
            pltpu.SemaphoreType.DMA((10,)),
        ],
        compiler_params=pltpu.CompilerParams(use_tc_tiling_on_sc=False),
    )(rows3d, zo)


NB = 5
NI = 10


def _agg_body(sh_hbm, rows_hbm, cols_hbm, zer_hbm, out_hbm,
              rb0, rb1, rb2, rb3, rb4,
              ri0, ri1, ri2, ri3, ri4, ri5, ri6, ri7, ri8, ri9,
              ci0, ci1, ci2, ci3, ci4, ci5, ci6, ci7, ci8, ci9,
              acc, gsem, ssem, rsem, csem):
    rb = (rb0, rb1, rb2, rb3, rb4)
    ri = (ri0, ri1, ri2, ri3, ri4, ri5, ri6, ri7, ri8, ri9)
    ci = (ci0, ci1, ci2, ci3, ci4, ci5, ci6, ci7, ci8, ci9)
    cid = lax.axis_index("c")
    sid = lax.axis_index("s")
    wid = cid * NSC + sid

    pltpu.sync_copy(zer_hbm, rb0)
    for t in range(ZB):
        pltpu.sync_copy(rb0, acc.at[pl.ds(sid * RPT + t * CH, CH)])
    plsc.subcore_barrier()

    def istart(m, s):
        pltpu.async_copy(rows_hbm.at[wid, m], ri[s], rsem.at[s])
        pltpu.async_copy(cols_hbm.at[wid, m], ci[s], csem.at[s])

    def iwait(m, s):
        pltpu.make_async_copy(rows_hbm.at[wid, m], ri[s], rsem.at[s]).wait()
        pltpu.make_async_copy(cols_hbm.at[wid, m], ci[s], csem.at[s]).wait()

    def gstart(s, si):
        pltpu.async_copy(sh_hbm.at[ci[si]], rb[s], gsem.at[s])

    def gwait(s, si):
        pltpu.make_async_copy(sh_hbm.at[ci[si]], rb[s], gsem.at[s]).wait()

    def sstart(s, si):
        pltpu.async_copy(rb[s], acc.at[ri[si]], ssem.at[s], add=True)

    def swait(s, si):
        pltpu.make_async_copy(rb[s], acc.at[ri[si]], ssem.at[s]).wait()

    def step(m, b, has_istart, has_g, has_swait):
        s, si = b % NB, b % NI
        if has_istart:
            istart(m + 4, (b + 4) % NI)
        if has_g:
            iwait(m + 2, (b + 2) % NI)
            if has_swait:
                swait((b + 2) % NB, (b - 3) % NI)
            gstart((b + 2) % NB, (b + 2) % NI)
        gwait(s, si)
        sstart(s, si)

    for m in range(4):
        istart(m, m)
    iwait(0, 0)
    gstart(0, 0)
    iwait(1, 1)
    gstart(1, 1)
    for m in range(10):
        step(m, m, has_istart=(m + 4 <= NCHUNK - 1), has_g=True,
             has_swait=(m >= 3))

    def kbody(k, carry):
        for b in range(10):
            step(10 * k + b, b, True, True, True)
        return carry

    lax.fori_loop(1, NCHUNK // 10 - 1, kbody, 0)

    base = NCHUNK - 10
    for b in range(10):
        m = base + b
        step(m, b, has_istart=(m + 4 <= NCHUNK - 1),
             has_g=(m + 2 <= NCHUNK - 1), has_swait=True)
    for m in range(NCHUNK - NB, NCHUNK):
        swait(m % NB, m % NI)

    plsc.subcore_barrier()
    pltpu.sync_copy(acc.at[pl.ds(sid * RPT, RPT)],
                    out_hbm.at[cid, pl.ds(sid * RPT, RPT)])


def _agg_kernel(sh2, rows3d, cols3d, zer):
    mesh = plsc.VectorSubcoreMesh(core_axis_name="c", subcore_axis_name="s")
    return pl.kernel(
        _agg_body,
        mesh=mesh,
        out_type=jax.ShapeDtypeStruct((NC, NP, DP), jnp.float32),
        scratch_types=(
            [pltpu.VMEM((CH, DP), jnp.float32)] * NB
            + [pltpu.VMEM((CH,), jnp.int32)] * (2 * NI)
            + [
                pltpu.VMEM_SHARED((NP, DP), jnp.float32),
                pltpu.SemaphoreType.DMA((NB,)),
                pltpu.SemaphoreType.DMA((NB,)),
                pltpu.SemaphoreType.DMA((NI,)),
                pltpu.SemaphoreType.DMA((NI,)),
            ]
        ),
        compiler_params=pltpu.CompilerParams(use_tc_tiling_on_sc=False),
    )(sh2, rows3d, cols3d, zer)


def _sphere_fix(v):
    n = jnp.sqrt(jnp.sum(v * v, axis=1, keepdims=True)) + EPS
    mask = (n > MAXN).astype(v.dtype)
    nc = jnp.minimum(n, MAXN)
    v_ = v / nc
    v_ = v_ * mask + v * (1 - mask)
    vn = jnp.sqrt(jnp.sum(v_ * v_, axis=1, keepdims=True))
    return v_ / jnp.clip(vn, 1e-12, None)


BN = 1000


def _stage1_body(x_ref, wb_ref, dg_ref, o_ref):
    xb = x_ref[...]
    wb = wb_ref[...]
    dg = dg_ref[...]
    deg = dg[:, 0:1] + dg[:, 1:2] + 1.0
    dis = lax.rsqrt(deg)

    u_t = xb[:, :T]
    u_sp = xb[:, T:]
    st2 = jnp.sum(u_t * u_t, axis=1, keepdims=True)
    ssp2 = jnp.sum(u_sp * u_sp, axis=1, keepdims=True)
    x0 = jnp.sqrt(jnp.clip(1.0 + ssp2 - st2, EPS, None))
    nt = jnp.sqrt(x0 * x0 + st2) + EPS
    f = jnp.concatenate(
        [x0 / nt, u_t / nt, u_sp,
         jnp.zeros((xb.shape[0], DP - 129), jnp.float32)], axis=1)
    fm = jnp.dot(f, wb, preferred_element_type=jnp.float32)
    s = fm[:, :T + 1]
    h = fm[:, T + 1:129]

    s_ = _sphere_fix(s)
    h2 = jnp.sum(h * h, axis=1, keepdims=True)
    h0 = jnp.sqrt(h2 + 1.0)

    gt = s_[:, 1:] * h0
    gt2 = jnp.sum(gt * gt, axis=1, keepdims=True)
    y0 = jnp.sqrt(jnp.clip(1.0 + h2 - gt2, EPS, None))
    nt2 = jnp.sqrt(y0 * y0 + gt2) + EPS
    out = jnp.concatenate(
        [y0 / nt2, gt / nt2, nt2, h,
         jnp.zeros((xb.shape[0], DP - D), jnp.float32)], axis=1)
    o_ref[...] = dis * out


def _stage1(x, wb, degp):
    grid = (N // BN,)
    return pl.pallas_call(
        _stage1_body,
        grid=grid,
        in_specs=[
            pl.BlockSpec((BN, 128), lambda i: (i, 0)),
            pl.BlockSpec((DP, DP), lambda i: (0, 0)),
            pl.BlockSpec((BN, NC), lambda i: (i, 0)),
        ],
        out_specs=pl.BlockSpec((BN, DP), lambda i: (i, 0)),
        out_shape=jax.ShapeDtypeStruct((N, DP), jnp.float32),
    )(x, wb, degp)


def _stage2_body(pp_ref, sh_ref, dg_ref, o_ref):
    pp = pp_ref[...]
    shr = sh_ref[...]
    dg = dg_ref[...]
    deg = dg[:, 0:1] + dg[:, 1:2] + 1.0
    dis = lax.rsqrt(deg)

    u = dis * (pp[0] + pp[1] + shr)
    s = u[:, :T + 1]
    s_ = _sphere_fix(s)

    ah0 = u[:, T + 1:T + 2]
    ahs = u[:, T + 2:D]
    mink = jnp.sum(ahs * ahs, axis=1, keepdims=True) - ah0 * ah0
    n = jnp.sqrt(jnp.abs(mink) + EPS) + EPS
    nc = jnp.minimum(n, MAXN)
    b0 = ah0 / nc
    bs = ahs / nc
    mink2 = jnp.sum(bs * bs, axis=1, keepdims=True) - b0 * b0
    n2 = jnp.sqrt(jnp.abs(mink2) + EPS) + EPS
    b0 = b0 / n2
    bs = bs / n2

    wt = s_[:, 1:] * b0
    wsp = bs

    wt2 = jnp.sum(wt * wt, axis=1, keepdims=True)
    wsp2 = jnp.sum(wsp * wsp, axis=1, keepdims=True)
    z0 = jnp.sqrt(jnp.clip(1.0 + wsp2 - wt2, EPS, None))
    ntz = jnp.sqrt(z0 * z0 + wt2) + EPS
    sz = jnp.concatenate([z0 / ntz, wt / ntz], axis=1)
    s2 = jnp.maximum(sz, 0.0)
    hsp = jnp.maximum(wsp, 0.0)

    s2_ = _sphere_fix(s2)
    hn = jnp.sqrt(jnp.sum(hsp * hsp, axis=1, keepdims=True)) + EPS
    maskh = (hn > MAXN).astype(jnp.float32)
    hnc = jnp.minimum(hn, MAXN)
    t = hsp / hnc
    tn = jnp.sqrt(jnp.sum(t * t, axis=1, keepdims=True))
    h_ = t / jnp.clip(tn, 1e-12, None) * MAXN
    hspf = h_ * maskh + hsp * (1 - maskh)
    h0z = jnp.sqrt(hnc * hnc + 1.0)

    o_ref[...] = jnp.concatenate([s2_[:, 1:] * h0z, hspf], axis=1)


def _stage2(parts, sh2, degp):
    grid = (N // BN,)
    return pl.pallas_call(
        _stage2_body,
        grid=grid,
        in_specs=[
            pl.BlockSpec((NC, BN, DP), lambda i: (0, i, 0)),
            pl.BlockSpec((BN, DP), lambda i: (i, 0)),
            pl.BlockSpec((BN, NC), lambda i: (i, 0)),
        ],
        out_specs=pl.BlockSpec((BN, 128), lambda i: (i, 0)),
        out_shape=jax.ShapeDtypeStruct((N, 128), jnp.float32),
    )(parts, sh2, degp)


def kernel(x, edge_index, W_time, W_space):
    rows3d = edge_index[0].reshape(NW, NCHUNK, CH)
    cols3d = edge_index[1].reshape(NW, NCHUNK, CH)
    zo = jnp.stack([jnp.zeros((CH,), jnp.float32),
                    jnp.ones((CH,), jnp.float32)])
    zer = jnp.zeros((CH, DP), jnp.float32)
    wb = jnp.zeros((DP, DP), jnp.float32)
    wb = wb.at[:T + 1, :T + 1].set(W_time)
    wb = wb.at[T + 1:129, T + 1:129].set(W_space)

    degp = _deg_kernel(rows3d, zo)
    dgn = degp[:, :N].T
    sh2 = _stage1(x, wb, dgn)
    parts = _agg_kernel(sh2, rows3d, cols3d, zer)
    return _stage2(parts, sh2, dgn)


# speedup vs baseline: 33.9668x; 1.0479x over previous
import functools

import jax
import jax.numpy as jnp
from jax import lax
from jax.experimental import pallas as pl
from jax.experimental.pallas import tpu as pltpu
from jax.experimental.pallas import tpu_sc as plsc

N = 10000
E = 320000
T = 7
S = 121
EPS = 1e-5
MAXN = 1e6
D = 130
DP = 144
NP = 10240
NC = 2
NSC = 16
NW = NC * NSC
EPW = E // NW
CH = 40
NCHUNK = EPW // CH
RPT = NP // NSC
ZB = RPT // CH


def _deg_body(e4_hbm, zo_hbm, out_hbm, idx2, zeros_v, ones_v, acc, ssem):
    cid = lax.axis_index("c")
    sid = lax.axis_index("s")
    wid = cid * NSC + sid
    pltpu.sync_copy(zo_hbm.at[0], zeros_v)
    pltpu.sync_copy(zo_hbm.at[1], ones_v)
    pltpu.sync_copy(e4_hbm.at[0, wid], idx2)

    def zbody(t, carry):
        pltpu.sync_copy(zeros_v, acc.at[pl.ds(sid * RPT + t * CH, CH)])
        return carry

    lax.fori_loop(0, ZB, zbody, 0)
    plsc.subcore_barrier()

    def sstart(j, s):
        pltpu.async_copy(ones_v, acc.at[idx2.at[j]], ssem.at[s], add=True)

    def swait(j, s):
        pltpu.make_async_copy(ones_v, acc.at[idx2.at[j]], ssem.at[s]).wait()

    for b in range(10):
        sstart(b, b)

    def body(k, carry):
        for b in range(10):
            j = 10 * k + b
            swait(j, b)
            sstart(j + 10, b)
        return carry

    lax.fori_loop(0, NCHUNK // 10 - 1, body, 0)
    for b in range(10):
        swait(NCHUNK - 10 + b, b)

    plsc.subcore_barrier()
    pltpu.sync_copy(acc.at[pl.ds(sid * RPT, RPT)],
                    out_hbm.at[cid, pl.ds(sid * RPT, RPT)])


def _deg_kernel(e4, zo):
    mesh = plsc.VectorSubcoreMesh(core_axis_name="c", subcore_axis_name="s")
    return pl.kernel(
        _deg_body,
        mesh=mesh,
        out_type=jax.ShapeDtypeStruct((NC, NP), jnp.float32),
        scratch_types=[
            pltpu.VMEM((NCHUNK, CH), jnp.int32),
            pltpu.VMEM((CH,), jnp.float32),
            pltpu.VMEM((CH,), jnp.float32),
            pltpu.VMEM_SHARED((NP,), jnp.float32),
            pltpu.SemaphoreType.DMA((10,)),
        ],
        compiler_params=pltpu.CompilerParams(use_tc_tiling_on_sc=False),
    )(e4, zo)


NB = 5
NI = 10


def _agg_body(sh_hbm, e4_hbm, zer_hbm, out_hbm,
              rb0, rb1, rb2, rb3, rb4,
              ri0, ri1, ri2, ri3, ri4, ri5, ri6, ri7, ri8, ri9,
              ci0, ci1, ci2, ci3, ci4, ci5, ci6, ci7, ci8, ci9,
              acc, gsem, ssem, rsem, csem):
    rb = (rb0, rb1, rb2, rb3, rb4)
    ri = (ri0, ri1, ri2, ri3, ri4, ri5, ri6, ri7, ri8, ri9)
    ci = (ci0, ci1, ci2, ci3, ci4, ci5, ci6, ci7, ci8, ci9)
    cid = lax.axis_index("c")
    sid = lax.axis_index("s")
    wid = cid * NSC + sid

    pltpu.sync_copy(zer_hbm, rb0)
    for t in range(ZB):
        pltpu.sync_copy(rb0, acc.at[pl.ds(sid * RPT + t * CH, CH)])
    plsc.subcore_barrier()

    def istart(m, s):
        pltpu.async_copy(e4_hbm.at[0, wid, m], ri[s], rsem.at[s])
        pltpu.async_copy(e4_hbm.at[1, wid, m], ci[s], csem.at[s])

    def iwait(m, s):
        pltpu.make_async_copy(e4_hbm.at[0, wid, m], ri[s], rsem.at[s]).wait()
        pltpu.make_async_copy(e4_hbm.at[1, wid, m], ci[s], csem.at[s]).wait()

    def gstart(s, si):
        pltpu.async_copy(sh_hbm.at[ci[si]], rb[s], gsem.at[s])

    def gwait(s, si):
        pltpu.make_async_copy(sh_hbm.at[ci[si]], rb[s], gsem.at[s]).wait()

    def sstart(s, si):
        pltpu.async_copy(rb[s], acc.at[ri[si]], ssem.at[s], add=True)

    def swait(s, si):
        pltpu.make_async_copy(rb[s], acc.at[ri[si]], ssem.at[s]).wait()

    def step(m, b, has_istart, has_g, has_swait):
        s, si = b % NB, b % NI
        if has_istart:
            istart(m + 4, (b + 4) % NI)
        if has_g:
            iwait(m + 2, (b + 2) % NI)
            if has_swait:
                swait((b + 2) % NB, (b - 3) % NI)
            gstart((b + 2) % NB, (b + 2) % NI)
        gwait(s, si)
        sstart(s, si)

    for m in range(4):
        istart(m, m)
    iwait(0, 0)
    gstart(0, 0)
    iwait(1, 1)
    gstart(1, 1)
    for m in range(10):
        step(m, m, has_istart=(m + 4 <= NCHUNK - 1), has_g=True,
             has_swait=(m >= 3))

    def kbody(k, carry):
        for b in range(10):
            step(10 * k + b, b, True, True, True)
        return carry

    lax.fori_loop(1, NCHUNK // 10 - 1, kbody, 0)

    base = NCHUNK - 10
    for b in range(10):
        m = base + b
        step(m, b, has_istart=(m + 4 <= NCHUNK - 1),
             has_g=(m + 2 <= NCHUNK - 1), has_swait=True)
    for m in range(NCHUNK - NB, NCHUNK):
        swait(m % NB, m % NI)

    plsc.subcore_barrier()
    pltpu.sync_copy(acc.at[pl.ds(sid * RPT, RPT)],
                    out_hbm.at[cid, pl.ds(sid * RPT, RPT)])


def _agg_kernel(sh2, e4, zer):
    mesh = plsc.VectorSubcoreMesh(core_axis_name="c", subcore_axis_name="s")
    return pl.kernel(
        _agg_body,
        mesh=mesh,
        out_type=jax.ShapeDtypeStruct((NC, NP, DP), jnp.float32),
        scratch_types=(
            [pltpu.VMEM((CH, DP), jnp.float32)] * NB
            + [pltpu.VMEM((CH,), jnp.int32)] * (2 * NI)
            + [
                pltpu.VMEM_SHARED((NP, DP), jnp.float32),
                pltpu.SemaphoreType.DMA((NB,)),
                pltpu.SemaphoreType.DMA((NB,)),
                pltpu.SemaphoreType.DMA((NI,)),
                pltpu.SemaphoreType.DMA((NI,)),
            ]
        ),
        compiler_params=pltpu.CompilerParams(use_tc_tiling_on_sc=False),
    )(sh2, e4, zer)


BN = 1000


def _fix_denom(s2sum):
    vnorm = jnp.sqrt(s2sum)
    n = vnorm + EPS
    mask = (n > MAXN).astype(jnp.float32)
    nc = jnp.minimum(n, MAXN)
    d_masked = nc * jnp.clip(vnorm / nc, 1e-12, None)
    d_plain = jnp.clip(vnorm, 1e-12, None)
    return d_masked * mask + d_plain * (1 - mask)


def _stage1a_body(x_ref, wb_ref, mx_ref, mf_ref, o_ref):
    xb = x_ref[...]
    wb = wb_ref[...]
    mx = mx_ref[...]
    mf = mf_ref[...]

    xx = jnp.dot(xb * xb, mx, preferred_element_type=jnp.float32)
    st2 = xx[:, 0:1]
    ssp2 = xx[:, 1:2]
    u_t = xb[:, :T]
    u_sp = xb[:, T:]
    x0 = jnp.sqrt(jnp.clip(1.0 + ssp2 - st2, EPS, None))
    nt = jnp.sqrt(x0 * x0 + st2) + EPS
    f = jnp.concatenate(
        [x0 / nt, u_t / nt, u_sp,
         jnp.zeros((xb.shape[0], DP - 129), jnp.float32)], axis=1)
    fm = jnp.dot(f, wb, preferred_element_type=jnp.float32)
    qq = jnp.dot(fm * fm, mf, preferred_element_type=jnp.float32)
    s2sum = qq[:, 0:1]
    h2 = qq[:, 1:2]

    s = fm[:, :T + 1]
    h = fm[:, T + 1:129]
    denom = _fix_denom(s2sum)
    h0 = jnp.sqrt(h2 + 1.0)

    gt = (s[:, 1:] / denom) * h0
    gt2 = (h0 * h0) * (s2sum - s[:, 0:1] * s[:, 0:1]) / (denom * denom)
    y0 = jnp.sqrt(jnp.clip(1.0 + h2 - gt2, EPS, None))
    nt2 = jnp.sqrt(y0 * y0 + gt2) + EPS
    o_ref[...] = jnp.concatenate(
        [y0 / nt2, gt / nt2, nt2, h,
         jnp.zeros((xb.shape[0], DP - D), jnp.float32)], axis=1)


def _stage1a(x, wb, mx, mf):
    grid = (N // BN,)
    return pl.pallas_call(
        _stage1a_body,
        grid=grid,
        in_specs=[
            pl.BlockSpec((BN, 128), lambda i: (i, 0)),
            pl.BlockSpec((DP, DP), lambda i: (0, 0)),
            pl.BlockSpec((128, 8), lambda i: (0, 0)),
            pl.BlockSpec((DP, 8), lambda i: (0, 0)),
        ],
        out_specs=pl.BlockSpec((BN, DP), lambda i: (i, 0)),
        out_shape=jax.ShapeDtypeStruct((N, DP), jnp.float32),
    )(x, wb, mx, mf)


def _stage1b_body(sh_ref, dg_ref, o_ref):
    dg = dg_ref[...]
    deg = dg[:, 0:1] + dg[:, 1:2] + 1.0
    o_ref[...] = lax.rsqrt(deg) * sh_ref[...]


def _stage1b(sh, degp):
    grid = (N // BN,)
    return pl.pallas_call(
        _stage1b_body,
        grid=grid,
        in_specs=[
            pl.BlockSpec((BN, DP), lambda i: (i, 0)),
            pl.BlockSpec((BN, NC), lambda i: (i, 0)),
        ],
        out_specs=pl.BlockSpec((BN, DP), lambda i: (i, 0)),
        out_shape=jax.ShapeDtypeStruct((N, DP), jnp.float32),
    )(sh, degp)


def _stage2_body(pp_ref, sh_ref, dg_ref, m2_ref, o_ref):
    pp = pp_ref[...]
    shr = sh_ref[...]
    dg = dg_ref[...]
    m2 = m2_ref[...]
    deg = dg[:, 0:1] + dg[:, 1:2] + 1.0
    dis = lax.rsqrt(deg)

    u = dis * (pp[0] + pp[1] + shr)
    up = jnp.maximum(u, 0.0)
    un = jnp.maximum(-u, 0.0)
    ru = jnp.dot(u * u, m2, preferred_element_type=jnp.float32)
    rp = jnp.dot(up * up, m2, preferred_element_type=jnp.float32)
    rn = jnp.dot(un * un, m2, preferred_element_type=jnp.float32)
    s2sum = ru[:, 0:1]
    ahs2 = ru[:, 1:2]
    sp17 = rp[:, 2:3]
    hp2 = rp[:, 3:4]
    sn17 = rn[:, 4:5]

    denom_s = _fix_denom(s2sum)
    u0 = u[:, 0:1]

    ah0 = u[:, T + 1:T + 2]
    ahs = u[:, T + 2:D]
    mink = ahs2 - ah0 * ah0
    n = jnp.sqrt(jnp.abs(mink) + EPS) + EPS
    nc = jnp.minimum(n, MAXN)
    mink2 = mink / (nc * nc)
    n2 = jnp.sqrt(jnp.abs(mink2) + EPS) + EPS
    ks = 1.0 / (nc * n2)
    b0 = ah0 * ks

    wt2 = (b0 * b0) * (s2sum - u0 * u0) / (denom_s * denom_s)
    wsp2 = ahs2 * (ks * ks)
    z0 = jnp.sqrt(jnp.clip(1.0 + wsp2 - wt2, EPS, None))
    ntz = jnp.sqrt(z0 * z0 + wt2) + EPS

    bpos = (b0 >= 0.0).astype(jnp.float32)
    relu_wt2 = (b0 * b0) * (bpos * sp17 + (1 - bpos) * sn17) / (denom_s * denom_s)
    s2sum2 = (z0 * z0 + relu_wt2) / (ntz * ntz)
    denom2 = _fix_denom(s2sum2)
    rwt = jnp.maximum(u[:, 1:T + 1] * b0, 0.0)
    hsp = ks * jnp.maximum(ahs, 0.0)

    hn0 = jnp.sqrt(hp2) * ks
    hn = hn0 + EPS
    maskh = (hn > MAXN).astype(jnp.float32)
    hnc = jnp.minimum(hn, MAXN)
    h_ = hsp / (hnc * jnp.clip(hn0 / hnc, 1e-12, None)) * MAXN
    hspf = h_ * maskh + hsp * (1 - maskh)
    h0z = jnp.sqrt(hnc * hnc + 1.0)

    s2tail = rwt / (denom_s * ntz * denom2) * h0z
    o_ref[...] = jnp.concatenate([s2tail, hspf], axis=1)


def _stage2(parts, sh2, degp, m2):
    grid = (N // BN,)
    return pl.pallas_call(
        _stage2_body,
        grid=grid,
        in_specs=[
            pl.BlockSpec((NC, BN, DP), lambda i: (0, i, 0)),
            pl.BlockSpec((BN, DP), lambda i: (i, 0)),
            pl.BlockSpec((BN, NC), lambda i: (i, 0)),
            pl.BlockSpec((DP, 8), lambda i: (0, 0)),
        ],
        out_specs=pl.BlockSpec((BN, 128), lambda i: (i, 0)),
        out_shape=jax.ShapeDtypeStruct((N, 128), jnp.float32),
    )(parts, sh2, degp, m2)


def kernel(x, edge_index, W_time, W_space):
    e4 = edge_index.reshape(2, NW, NCHUNK, CH)
    zo = jnp.stack([jnp.zeros((CH,), jnp.float32),
                    jnp.ones((CH,), jnp.float32)])
    zer = jnp.zeros((CH, DP), jnp.float32)
    wb = jnp.zeros((DP, DP), jnp.float32)
    wb = wb.at[:T + 1, :T + 1].set(W_time)
    wb = wb.at[T + 1:129, T + 1:129].set(W_space)
    mx = jnp.zeros((128, 8), jnp.float32)
    mx = mx.at[:T, 0].set(1.0).at[T:, 1].set(1.0)
    mf = jnp.zeros((DP, 8), jnp.float32)
    mf = mf.at[:T + 1, 0].set(1.0).at[T + 1:129, 1].set(1.0)
    m2 = jnp.zeros((DP, 8), jnp.float32)
    m2 = (m2.at[:T + 1, 0].set(1.0)
            .at[T + 2:D, 1].set(1.0)
            .at[1:T + 1, 2].set(1.0)
            .at[T + 2:D, 3].set(1.0)
            .at[1:T + 1, 4].set(1.0))

    degp = _deg_kernel(e4, zo)
    dgn = degp[:, :N].T
    sh = _stage1a(x, wb, mx, mf)
    sh2 = _stage1b(sh, dgn)
    parts = _agg_kernel(sh2, e4, zer)
    return _stage2(parts, sh2, dgn, m2)


# speedup vs baseline: 35.1295x; 1.0342x over previous
import functools

import jax
import jax.numpy as jnp
from jax import lax
from jax.experimental import pallas as pl
from jax.experimental.pallas import tpu as pltpu
from jax.experimental.pallas import tpu_sc as plsc

N = 10000
E = 320000
T = 7
S = 121
EPS = 1e-5
MAXN = 1e6
D = 130
DP = 144
NP = 10240
NC = 2
NSC = 16
NW = NC * NSC
EPW = E // NW
CH = 40
NCHUNK = EPW // CH
RPT = NP // NSC
ZB = RPT // CH


def _deg_body(e4_hbm, zo_hbm, out_hbm, idx2, zeros_v, ones_v, acc, ssem):
    cid = lax.axis_index("c")
    sid = lax.axis_index("s")
    wid = cid * NSC + sid
    pltpu.sync_copy(zo_hbm.at[0], zeros_v)
    pltpu.sync_copy(zo_hbm.at[1], ones_v)
    pltpu.sync_copy(e4_hbm.at[0, wid], idx2)

    def zbody(t, carry):
        pltpu.sync_copy(zeros_v, acc.at[pl.ds(sid * RPT + t * CH, CH)])
        return carry

    lax.fori_loop(0, ZB, zbody, 0)
    plsc.subcore_barrier()

    def sstart(j, s):
        pltpu.async_copy(ones_v, acc.at[idx2.at[j]], ssem.at[s], add=True)

    def swait(j, s):
        pltpu.make_async_copy(ones_v, acc.at[idx2.at[j]], ssem.at[s]).wait()

    for b in range(10):
        sstart(b, b)

    def body(k, carry):
        for b in range(10):
            j = 10 * k + b
            swait(j, b)
            sstart(j + 10, b)
        return carry

    lax.fori_loop(0, NCHUNK // 10 - 1, body, 0)
    for b in range(10):
        swait(NCHUNK - 10 + b, b)

    plsc.subcore_barrier()
    pltpu.sync_copy(acc.at[pl.ds(sid * RPT, RPT)],
                    out_hbm.at[cid, pl.ds(sid * RPT, RPT)])


def _deg_kernel(e4, zo):
    mesh = plsc.VectorSubcoreMesh(core_axis_name="c", subcore_axis_name="s")
    return pl.kernel(
        _deg_body,
        mesh=mesh,
        out_type=jax.ShapeDtypeStruct((NC, NP), jnp.float32),
        scratch_types=[
            pltpu.VMEM((NCHUNK, CH), jnp.int32),
            pltpu.VMEM((CH,), jnp.float32),
            pltpu.VMEM((CH,), jnp.float32),
            pltpu.VMEM_SHARED((NP,), jnp.float32),
            pltpu.SemaphoreType.DMA((10,)),
        ],
        compiler_params=pltpu.CompilerParams(use_tc_tiling_on_sc=False),
    )(e4, zo)


NB = 5
NI = 10


def _agg_body(sh_hbm, e4_hbm, zer_hbm, out_hbm,
              rb0, rb1, rb2, rb3, rb4,
              ri0, ri1, ri2, ri3, ri4, ri5, ri6, ri7, ri8, ri9,
              ci0, ci1, ci2, ci3, ci4, ci5, ci6, ci7, ci8, ci9,
              acc, gsem, ssem, rsem, csem):
    rb = (rb0, rb1, rb2, rb3, rb4)
    ri = (ri0, ri1, ri2, ri3, ri4, ri5, ri6, ri7, ri8, ri9)
    ci = (ci0, ci1, ci2, ci3, ci4, ci5, ci6, ci7, ci8, ci9)
    cid = lax.axis_index("c")
    sid = lax.axis_index("s")
    wid = cid * NSC + sid

    pltpu.sync_copy(zer_hbm, rb0)
    for t in range(ZB):
        pltpu.sync_copy(rb0, acc.at[pl.ds(sid * RPT + t * CH, CH)])
    plsc.subcore_barrier()

    def istart(m, s):
        pltpu.async_copy(e4_hbm.at[0, wid, m], ri[s], rsem.at[s])
        pltpu.async_copy(e4_hbm.at[1, wid, m], ci[s], csem.at[s])

    def iwait(m, s):
        pltpu.make_async_copy(e4_hbm.at[0, wid, m], ri[s], rsem.at[s]).wait()
        pltpu.make_async_copy(e4_hbm.at[1, wid, m], ci[s], csem.at[s]).wait()

    def gstart(s, si):
        pltpu.async_copy(sh_hbm.at[ci[si]], rb[s], gsem.at[s])

    def gwait(s, si):
        pltpu.make_async_copy(sh_hbm.at[ci[si]], rb[s], gsem.at[s]).wait()

    def sstart(s, si):
        pltpu.async_copy(rb[s], acc.at[ri[si]], ssem.at[s], add=True)

    def swait(s, si):
        pltpu.make_async_copy(rb[s], acc.at[ri[si]], ssem.at[s]).wait()

    def step(m, b, has_istart, has_g, has_swait):
        s, si = b % NB, b % NI
        if has_istart:
            istart(m + 4, (b + 4) % NI)
        if has_g:
            iwait(m + 2, (b + 2) % NI)
            if has_swait:
                swait((b + 2) % NB, (b - 3) % NI)
            gstart((b + 2) % NB, (b + 2) % NI)
        gwait(s, si)
        sstart(s, si)

    for m in range(4):
        istart(m, m)
    iwait(0, 0)
    gstart(0, 0)
    iwait(1, 1)
    gstart(1, 1)
    for m in range(10):
        step(m, m, has_istart=(m + 4 <= NCHUNK - 1), has_g=True,
             has_swait=(m >= 3))

    def kbody(k, carry):
        for b in range(10):
            step(10 * k + b, b, True, True, True)
        return carry

    lax.fori_loop(1, NCHUNK // 10 - 1, kbody, 0)

    base = NCHUNK - 10
    for b in range(10):
        m = base + b
        step(m, b, has_istart=(m + 4 <= NCHUNK - 1),
             has_g=(m + 2 <= NCHUNK - 1), has_swait=True)
    for m in range(NCHUNK - NB, NCHUNK):
        swait(m % NB, m % NI)

    plsc.subcore_barrier()
    pltpu.sync_copy(acc.at[pl.ds(sid * RPT, RPT)],
                    out_hbm.at[cid, pl.ds(sid * RPT, RPT)])


def _agg_kernel(sh2, e4, zer):
    mesh = plsc.VectorSubcoreMesh(core_axis_name="c", subcore_axis_name="s")
    return pl.kernel(
        _agg_body,
        mesh=mesh,
        out_type=jax.ShapeDtypeStruct((NC, NP, DP), jnp.float32),
        scratch_types=(
            [pltpu.VMEM((CH, DP), jnp.float32)] * NB
            + [pltpu.VMEM((CH,), jnp.int32)] * (2 * NI)
            + [
                pltpu.VMEM_SHARED((NP, DP), jnp.float32),
                pltpu.SemaphoreType.DMA((NB,)),
                pltpu.SemaphoreType.DMA((NB,)),
                pltpu.SemaphoreType.DMA((NI,)),
                pltpu.SemaphoreType.DMA((NI,)),
            ]
        ),
        compiler_params=pltpu.CompilerParams(use_tc_tiling_on_sc=False),
    )(sh2, e4, zer)


BN = 2000


def _fix_denom(s2sum):
    vnorm = jnp.sqrt(s2sum)
    n = vnorm + EPS
    mask = (n > MAXN).astype(jnp.float32)
    nc = jnp.minimum(n, MAXN)
    d_masked = nc * jnp.clip(vnorm / nc, 1e-12, None)
    d_plain = jnp.clip(vnorm, 1e-12, None)
    return d_masked * mask + d_plain * (1 - mask)


def _stage1_body(x_ref, wb_ref, mx_ref, mf_ref, dg_ref, o_ref):
    xb = x_ref[...]
    wb = wb_ref[...]
    mx = mx_ref[...]
    mf = mf_ref[...]
    dg = dg_ref[...]
    deg = dg[:, 0:1] + dg[:, 1:2] + 1.0
    dis = lax.rsqrt(deg)

    xx = jnp.dot(xb * xb, mx, preferred_element_type=jnp.float32)
    st2 = xx[:, 0:1]
    ssp2 = xx[:, 1:2]
    u_t = xb[:, :T]
    u_sp = xb[:, T:]
    x0 = jnp.sqrt(jnp.clip(1.0 + ssp2 - st2, EPS, None))
    nt = jnp.sqrt(x0 * x0 + st2) + EPS
    f = jnp.concatenate(
        [x0 / nt, u_t / nt, u_sp,
         jnp.zeros((xb.shape[0], DP - 129), jnp.float32)], axis=1)
    fm = jnp.dot(f, wb, preferred_element_type=jnp.float32)
    qq = jnp.dot(fm * fm, mf, preferred_element_type=jnp.float32)
    s2sum = qq[:, 0:1]
    h2 = qq[:, 1:2]

    s = fm[:, :T + 1]
    h = fm[:, T + 1:129]
    denom = _fix_denom(s2sum)
    h0 = jnp.sqrt(h2 + 1.0)

    gt = (s[:, 1:] / denom) * h0
    gt2 = (h0 * h0) * (s2sum - s[:, 0:1] * s[:, 0:1]) / (denom * denom)
    y0 = jnp.sqrt(jnp.clip(1.0 + h2 - gt2, EPS, None))
    nt2 = jnp.sqrt(y0 * y0 + gt2) + EPS
    o_ref[...] = dis * jnp.concatenate(
        [y0 / nt2, gt / nt2, nt2, h,
         jnp.zeros((xb.shape[0], DP - D), jnp.float32)], axis=1)


def _stage1(x, wb, mx, mf, degp):
    grid = (N // BN,)
    return pl.pallas_call(
        _stage1_body,
        grid=grid,
        in_specs=[
            pl.BlockSpec((BN, 128), lambda i: (i, 0)),
            pl.BlockSpec((DP, DP), lambda i: (0, 0)),
            pl.BlockSpec((128, 8), lambda i: (0, 0)),
            pl.BlockSpec((DP, 8), lambda i: (0, 0)),
            pl.BlockSpec((BN, NC), lambda i: (i, 0)),
        ],
        out_specs=pl.BlockSpec((BN, DP), lambda i: (i, 0)),
        out_shape=jax.ShapeDtypeStruct((N, DP), jnp.float32),
    )(x, wb, mx, mf, degp)


def _stage2_body(pp_ref, sh_ref, dg_ref, m2_ref, o_ref):
    pp = pp_ref[...]
    shr = sh_ref[...]
    dg = dg_ref[...]
    m2 = m2_ref[...]
    deg = dg[:, 0:1] + dg[:, 1:2] + 1.0
    dis = lax.rsqrt(deg)

    u = dis * (pp[0] + pp[1] + shr)
    up = jnp.maximum(u, 0.0)
    un17 = jnp.minimum(u[:, 1:T + 1], 0.0)
    ru = jnp.dot(u * u, m2, preferred_element_type=jnp.float32)
    rp = jnp.dot(up * up, m2, preferred_element_type=jnp.float32)
    s2sum = ru[:, 0:1]
    ahs2 = ru[:, 1:2]
    sp17 = rp[:, 2:3]
    hp2 = rp[:, 3:4]
    sn17 = jnp.sum(un17 * un17, axis=1, keepdims=True)

    denom_s = _fix_denom(s2sum)
    u0 = u[:, 0:1]

    ah0 = u[:, T + 1:T + 2]
    ahs = u[:, T + 2:D]
    mink = ahs2 - ah0 * ah0
    n = jnp.sqrt(jnp.abs(mink) + EPS) + EPS
    nc = jnp.minimum(n, MAXN)
    mink2 = mink / (nc * nc)
    n2 = jnp.sqrt(jnp.abs(mink2) + EPS) + EPS
    ks = 1.0 / (nc * n2)
    b0 = ah0 * ks

    wt2 = (b0 * b0) * (s2sum - u0 * u0) / (denom_s * denom_s)
    wsp2 = ahs2 * (ks * ks)
    z0 = jnp.sqrt(jnp.clip(1.0 + wsp2 - wt2, EPS, None))
    ntz = jnp.sqrt(z0 * z0 + wt2) + EPS

    bpos = (b0 >= 0.0).astype(jnp.float32)
    relu_wt2 = (b0 * b0) * (bpos * sp17 + (1 - bpos) * sn17) / (denom_s * denom_s)
    s2sum2 = (z0 * z0 + relu_wt2) / (ntz * ntz)
    denom2 = _fix_denom(s2sum2)
    rwt = jnp.maximum(u[:, 1:T + 1] * b0, 0.0)
    hsp = ks * jnp.maximum(ahs, 0.0)

    hn0 = jnp.sqrt(hp2) * ks
    hn = hn0 + EPS
    maskh = (hn > MAXN).astype(jnp.float32)
    hnc = jnp.minimum(hn, MAXN)
    h_ = hsp / (hnc * jnp.clip(hn0 / hnc, 1e-12, None)) * MAXN
    hspf = h_ * maskh + hsp * (1 - maskh)
    h0z = jnp.sqrt(hnc * hnc + 1.0)

    s2tail = rwt / (denom_s * ntz * denom2) * h0z
    o_ref[...] = jnp.concatenate([s2tail, hspf], axis=1)


def _stage2(parts, sh2, degp, m2):
    grid = (N // BN,)
    return pl.pallas_call(
        _stage2_body,
        grid=grid,
        in_specs=[
            pl.BlockSpec((NC, BN, DP), lambda i: (0, i, 0)),
            pl.BlockSpec((BN, DP), lambda i: (i, 0)),
            pl.BlockSpec((BN, NC), lambda i: (i, 0)),
            pl.BlockSpec((DP, 8), lambda i: (0, 0)),
        ],
        out_specs=pl.BlockSpec((BN, 128), lambda i: (i, 0)),
        out_shape=jax.ShapeDtypeStruct((N, 128), jnp.float32),
    )(parts, sh2, degp, m2)


def kernel(x, edge_index, W_time, W_space):
    e4 = edge_index.reshape(2, NW, NCHUNK, CH)
    zo = jnp.stack([jnp.zeros((CH,), jnp.float32),
                    jnp.ones((CH,), jnp.float32)])
    zer = jnp.zeros((CH, DP), jnp.float32)
    wb = jnp.zeros((DP, DP), jnp.float32)
    wb = wb.at[:T + 1, :T + 1].set(W_time)
    wb = wb.at[T + 1:129, T + 1:129].set(W_space)
    mx = jnp.zeros((128, 8), jnp.float32)
    mx = mx.at[:T, 0].set(1.0).at[T:, 1].set(1.0)
    mf = jnp.zeros((DP, 8), jnp.float32)
    mf = mf.at[:T + 1, 0].set(1.0).at[T + 1:129, 1].set(1.0)
    m2 = jnp.zeros((DP, 8), jnp.float32)
    m2 = (m2.at[:T + 1, 0].set(1.0)
            .at[T + 2:D, 1].set(1.0)
            .at[1:T + 1, 2].set(1.0)
            .at[T + 2:D, 3].set(1.0)
            .at[1:T + 1, 4].set(1.0))

    degp = _deg_kernel(e4, zo)
    dgn = degp[:, :N].T
    sh2 = _stage1(x, wb, mx, mf, dgn)
    parts = _agg_kernel(sh2, e4, zer)
    return _stage2(parts, sh2, dgn, m2)
